# Initial kernel scaffold; baseline (speedup 1.0000x reference)
import jax
import jax.numpy as jnp
from jax.experimental import pallas as pl


def kernel(x, d_weight, h_weight, w_weight):
    raise NotImplementedError("write your pallas kernel here")


# speedup vs baseline: 1.0416x; 1.0416x over previous
import jax
import jax.numpy as jnp
from jax.experimental import pallas as pl


def _body(dpt_ref, hpt_ref, wpt_ref, out_ref):
    cb, d = dpt_ref.shape
    hw = hpt_ref.shape[1] * wpt_ref.shape[1]
    h = hpt_ref[...]
    w = wpt_ref[...]
    hwsum = (h[:, :, None] + w[:, None, :]).reshape(cb, hw)
    dv = dpt_ref[...]
    for i in range(d):
        out_ref[0, :, i, :] = hwsum + dv[:, i][:, None]


def kernel(x, d_weight, h_weight, w_weight):
    B = x.shape[0]
    d, h, w = x.shape[-3:]
    c = d_weight.shape[1]
    C = 256

    f32 = jnp.float32
    dpt = jnp.zeros((C, d), f32).at[0:c, :].set(d_weight[:d].T.astype(f32))
    hpt = jnp.zeros((C, h), f32).at[c:2 * c, :].set(h_weight[:h].T.astype(f32))
    wpt = jnp.zeros((C, w), f32).at[2 * c:C, :].set(
        w_weight[:w, : C - 2 * c].T.astype(f32))

    CB = 16
    grid = (B, C // CB)
    out4 = pl.pallas_call(
        _body,
        grid=grid,
        in_specs=[
            pl.BlockSpec((CB, d), lambda b, pc: (pc, 0)),
            pl.BlockSpec((CB, h), lambda b, pc: (pc, 0)),
            pl.BlockSpec((CB, w), lambda b, pc: (pc, 0)),
        ],
        out_specs=pl.BlockSpec((1, CB, d, h * w), lambda b, pc: (b, pc, 0, 0)),
        out_shape=jax.ShapeDtypeStruct((B, C, d, h * w), f32),
    )(dpt, hpt, wpt)
    return out4.reshape(B, C, d, h, w)


# speedup vs baseline: 1.1521x; 1.1061x over previous
import jax
import jax.numpy as jnp
from jax.experimental import pallas as pl


def _body(dpt_ref, hpt_ref, wpt_ref, out_ref):
    nb = out_ref.shape[0]
    cb, d = dpt_ref.shape
    hw = hpt_ref.shape[1] * wpt_ref.shape[1]
    h = hpt_ref[...]
    w = wpt_ref[...]
    hwsum = (h[:, :, None] + w[:, None, :]).reshape(cb, hw)
    dv = dpt_ref[...]
    for i in range(d):
        row = hwsum + dv[:, i][:, None]
        for b in range(nb):
            out_ref[b, :, i, :] = row


def kernel(x, d_weight, h_weight, w_weight):
    B = x.shape[0]
    d, h, w = x.shape[-3:]
    c = d_weight.shape[1]
    C = 256

    f32 = jnp.float32
    dpt = jnp.zeros((C, d), f32).at[0:c, :].set(d_weight[:d].T.astype(f32))
    hpt = jnp.zeros((C, h), f32).at[c:2 * c, :].set(h_weight[:h].T.astype(f32))
    wpt = jnp.zeros((C, w), f32).at[2 * c:C, :].set(
        w_weight[:w, : C - 2 * c].T.astype(f32))

    CB = 16
    grid = (C // CB,)
    out4 = pl.pallas_call(
        _body,
        grid=grid,
        in_specs=[
            pl.BlockSpec((CB, d), lambda pc: (pc, 0)),
            pl.BlockSpec((CB, h), lambda pc: (pc, 0)),
            pl.BlockSpec((CB, w), lambda pc: (pc, 0)),
        ],
        out_specs=pl.BlockSpec((B, CB, d, h * w), lambda pc: (0, pc, 0, 0)),
        out_shape=jax.ShapeDtypeStruct((B, C, d, h * w), f32),
    )(dpt, hpt, wpt)
    return out4.reshape(B, C, d, h, w)


# speedup vs baseline: 1.1711x; 1.0166x over previous
import jax
import jax.numpy as jnp
from jax.experimental import pallas as pl


def _body(dpt_ref, hpt_ref, wpt_ref, out_ref):
    nb = out_ref.shape[0]
    cb, d = dpt_ref.shape
    hw = hpt_ref.shape[1] * wpt_ref.shape[1]
    h = hpt_ref[...]
    w = wpt_ref[...]
    hwsum = (h[:, :, None] + w[:, None, :]).reshape(cb, hw)
    dv = dpt_ref[...]
    for i in range(d):
        row = hwsum + dv[:, i][:, None]
        for b in range(nb):
            out_ref[b, :, i, :] = row


def kernel(x, d_weight, h_weight, w_weight):
    B = x.shape[0]
    d, h, w = x.shape[-3:]
    c = d_weight.shape[1]
    C = 256

    f32 = jnp.float32
    dpt = jnp.zeros((C, d), f32).at[0:c, :].set(d_weight[:d].T.astype(f32))
    hpt = jnp.zeros((C, h), f32).at[c:2 * c, :].set(h_weight[:h].T.astype(f32))
    wpt = jnp.zeros((C, w), f32).at[2 * c:C, :].set(
        w_weight[:w, : C - 2 * c].T.astype(f32))

    CB = 32
    grid = (C // CB,)
    out4 = pl.pallas_call(
        _body,
        grid=grid,
        in_specs=[
            pl.BlockSpec((CB, d), lambda pc: (pc, 0)),
            pl.BlockSpec((CB, h), lambda pc: (pc, 0)),
            pl.BlockSpec((CB, w), lambda pc: (pc, 0)),
        ],
        out_specs=pl.BlockSpec((B, CB, d, h * w), lambda pc: (0, pc, 0, 0)),
        out_shape=jax.ShapeDtypeStruct((B, C, d, h * w), f32),
    )(dpt, hpt, wpt)
    return out4.reshape(B, C, d, h, w)
